# baseline (speedup 1.0000x reference)
import functools

import jax
import jax.numpy as jnp
from jax import lax
from jax.experimental import pallas as pl
from jax.experimental.pallas import tpu as pltpu

_CH = 32
_NEG = -1e30
_POS = 1e30


def _body(x1_ref, w1t_ref, g1_ref, b1_ref, m1_ref, v1_ref,
          w2t_ref, g2_ref, b2_ref, m2_ref, v2_ref, wlt_ref, bl_ref,
          out_ref):
    f32 = jnp.float32

    def chunk(i, hall):
        l = x1_ref[0, 0, pl.ds(i * _CH, _CH), :]
        a = x1_ref[0, 1, pl.ds(i * _CH, _CH), :]
        b = x1_ref[0, 2, pl.ds(i * _CH, _CH), :]
        va = (a + 1.0) * 0.5
        vb = (b + 1.0) * 0.5
        vl = (l + 1.0) * 0.5
        wab = ((a != 0) & (b != 0) & (va >= 0) & (va <= 1)
               & (vb >= 0) & (vb <= 1)).astype(jnp.bfloat16)
        wlm = ((l != 0) & (vl >= 0) & (vl <= 1)).astype(jnp.bfloat16)
        ia = jnp.clip(jnp.floor(va * 64.0).astype(jnp.int32), 0, 63)
        ib = jnp.clip(jnp.floor(vb * 64.0).astype(jnp.int32), 0, 63)
        il = jnp.clip(jnp.floor(vl * 8.0).astype(jnp.int32), 0, 7)
        bins64 = lax.broadcasted_iota(jnp.int32, (_CH, 512, 64), 2)
        bins8 = lax.broadcasted_iota(jnp.int32, (_CH, 512, 8), 2)
        oa = (ia[:, :, None] == bins64).astype(jnp.bfloat16)
        ob = (ib[:, :, None] == bins64).astype(jnp.bfloat16) * wab[:, :, None]
        olw = (il[:, :, None] == bins8).astype(jnp.bfloat16) * wlm[:, :, None]
        lhs = jnp.concatenate(
            [ob, jnp.ones((_CH, 512, 1), jnp.bfloat16)], 2).reshape(_CH * 512, 65)
        rhs = jnp.concatenate([oa, olw], 2).reshape(_CH * 512, 72)
        return hall + lax.dot_general(lhs, rhs,
                                      (((0,), (0,)), ((), ())),
                                      preferred_element_type=f32)

    hall = lax.fori_loop(0, 512 // _CH, chunk, jnp.zeros((65, 72), f32))
    h2 = hall[0:64, 0:64]
    h2 = h2 / jnp.sum(h2)
    h1 = hall[64:65, 64:72]
    h1 = h1 / jnp.sum(h1)

    inv1 = g1_ref[...] * lax.rsqrt(v1_ref[...] + 1e-5)
    sh1 = b1_ref[...] - m1_ref[...] * inv1
    amul = w1t_ref[0:1, :] * inv1
    cadd = jnp.dot(h1, w1t_ref[1:9, :],
                   preferred_element_type=f32) * inv1 + sh1

    def pad2(m, val):
        col = jnp.full((64, 1), val, f32)
        mc = jnp.concatenate([col, m, col], axis=1)
        row = jnp.full((1, 66), val, f32)
        return jnp.concatenate([row, mc, row], axis=0)

    def run3(m, val, op):
        row = jnp.full((1, 66), val, f32)
        mr = jnp.concatenate([row, m, row], axis=0)
        m = op(op(mr[0:66], mr[1:67]), mr[2:68])
        col = jnp.full((66, 1), val, f32)
        mc = jnp.concatenate([col, m, col], axis=1)
        return op(op(mc[:, 0:66], mc[:, 1:67]), mc[:, 2:68])

    mx = run3(pad2(h2, _NEG), _NEG, jnp.maximum)
    mn = run3(pad2(h2, _POS), _POS, jnp.minimum)
    s1 = (lax.broadcasted_iota(jnp.int32, (33, 66), 1)
          == 2 * lax.broadcasted_iota(jnp.int32, (33, 66), 0)).astype(f32)
    s2 = (lax.broadcasted_iota(jnp.int32, (66, 33), 0)
          == 2 * lax.broadcasted_iota(jnp.int32, (66, 33), 1)).astype(f32)
    pmax = jnp.dot(jnp.dot(s1, mx, preferred_element_type=f32), s2,
                   preferred_element_type=f32)
    pmin = jnp.dot(jnp.dot(s1, mn, preferred_element_type=f32), s2,
                   preferred_element_type=f32)

    apos = jnp.maximum(amul, 0.0)
    aneg = jnp.minimum(amul, 0.0)
    astk = jnp.concatenate([apos, aneg], axis=0)
    zs = []
    for j in range(33):
        cols = jnp.concatenate([pmax[:, j:j + 1], pmin[:, j:j + 1]], axis=1)
        zs.append(jnp.dot(cols, astk, preferred_element_type=f32) + cadd)
    zint = jnp.stack(zs, axis=0)

    ii = lax.broadcasted_iota(jnp.int32, (33, 33, 128), 1)
    jj = lax.broadcasted_iota(jnp.int32, (33, 33, 128), 0)
    bflag = (ii == 0) | (ii == 32) | (jj == 0) | (jj == 32)
    sh3 = sh1.reshape(1, 1, 128)
    bterm = jnp.where(bflag, sh3, _NEG)
    z = jnp.maximum(jnp.maximum(zint, bterm), 0.0)

    zb = z.astype(jnp.bfloat16)
    zc0 = jnp.zeros((33, 1, 128), jnp.bfloat16)
    zc = jnp.concatenate([zc0, zb, zc0], axis=1)
    zr0 = jnp.zeros((1, 35, 128), jnp.bfloat16)
    xp = jnp.concatenate([zr0, zc, zr0], axis=0)

    xs = [xp[dj:dj + 33, di:di + 33, :].reshape(33 * 33, 128)
          for di in range(3) for dj in range(3)]
    xcat = jnp.concatenate(xs, axis=1)
    acc = jnp.dot(xcat, w2t_ref[...], preferred_element_type=f32)

    inv2 = g2_ref[...] * lax.rsqrt(v2_ref[...] + 1e-5)
    sh2 = b2_ref[...] - m2_ref[...] * inv2
    y2 = jnp.maximum(acc * inv2 + sh2, 0.0)
    mv = jnp.sum(y2, axis=0, keepdims=True) * (1.0 / 1089.0)
    hv = jnp.dot(mv, wlt_ref[...], preferred_element_type=f32) + bl_ref[...]
    out_ref[...] = hv.reshape(1, 1, 256)


@functools.partial(jax.jit)
def kernel(x, x1, w1, g1, b1, m1, v1, w2, g2, b2, m2, v2, wl, bl):
    B = x1.shape[0]
    w1t = w1.reshape(128, 9).T
    w2t = jnp.transpose(w2, (2, 3, 1, 0)).reshape(9 * 128, 1024)
    w2t = w2t.astype(jnp.bfloat16)
    wlt = wl.T
    row = lambda v: v.reshape(1, -1)

    h = pl.pallas_call(
        _body,
        grid=(B,),
        in_specs=[
            pl.BlockSpec((1, 3, 512, 512), lambda b: (b, 0, 0, 0)),
            pl.BlockSpec((9, 128), lambda b: (0, 0)),
            pl.BlockSpec((1, 128), lambda b: (0, 0)),
            pl.BlockSpec((1, 128), lambda b: (0, 0)),
            pl.BlockSpec((1, 128), lambda b: (0, 0)),
            pl.BlockSpec((1, 128), lambda b: (0, 0)),
            pl.BlockSpec((9 * 128, 1024), lambda b: (0, 0)),
            pl.BlockSpec((1, 1024), lambda b: (0, 0)),
            pl.BlockSpec((1, 1024), lambda b: (0, 0)),
            pl.BlockSpec((1, 1024), lambda b: (0, 0)),
            pl.BlockSpec((1, 1024), lambda b: (0, 0)),
            pl.BlockSpec((1024, 256), lambda b: (0, 0)),
            pl.BlockSpec((1, 256), lambda b: (0, 0)),
        ],
        out_specs=pl.BlockSpec((1, 1, 256), lambda b: (b, 0, 0)),
        out_shape=jax.ShapeDtypeStruct((B, 1, 256), jnp.float32),
        compiler_params=pltpu.CompilerParams(
            dimension_semantics=("parallel",)),
    )(x1, w1t, row(g1), row(b1), row(m1), row(v1),
      w2t, row(g2), row(b2), row(m2), row(v2), wlt, row(bl))
    return jnp.concatenate([x, h.reshape(B, 256)], axis=1)


# speedup vs baseline: 1.0279x; 1.0279x over previous
import functools

import jax
import jax.numpy as jnp
from jax import lax
from jax.experimental import pallas as pl
from jax.experimental.pallas import tpu as pltpu

_CH = 32
_NEG = -1e30
_POS = 1e30


def _body(x1_ref, w1t_ref, g1_ref, b1_ref, m1_ref, v1_ref,
          w2t_ref, g2_ref, b2_ref, m2_ref, v2_ref, wlt_ref, bl_ref,
          out_ref):
    f32 = jnp.float32

    def chunk(i, hall):
        l = x1_ref[0, 0, pl.ds(i * _CH, _CH), :]
        a = x1_ref[0, 1, pl.ds(i * _CH, _CH), :]
        b = x1_ref[0, 2, pl.ds(i * _CH, _CH), :]
        va = (a + 1.0) * 0.5
        vb = (b + 1.0) * 0.5
        vl = (l + 1.0) * 0.5
        wab = ((a != 0) & (b != 0) & (va >= 0) & (va <= 1)
               & (vb >= 0) & (vb <= 1)).astype(jnp.bfloat16)
        wlm = ((l != 0) & (vl >= 0) & (vl <= 1)).astype(jnp.bfloat16)
        ia = jnp.clip(jnp.floor(va * 64.0).astype(jnp.int32), 0, 63)
        ib = jnp.clip(jnp.floor(vb * 64.0).astype(jnp.int32), 0, 63)
        il = jnp.clip(jnp.floor(vl * 8.0).astype(jnp.int32), 0, 7)
        h2, h1 = hall
        bins64 = lax.broadcasted_iota(jnp.int32, (_CH, 512, 64), 2)
        bins8 = lax.broadcasted_iota(jnp.int32, (_CH, 512, 8), 2)
        oa = (ia[:, :, None] == bins64).astype(jnp.bfloat16)
        ob = (ib[:, :, None] == bins64).astype(jnp.bfloat16) * wab[:, :, None]
        oa2 = oa.reshape(_CH * 512, 64)
        ob2 = ob.reshape(_CH * 512, 64)
        h2 = h2 + lax.dot_general(ob2, oa2, (((0,), (0,)), ((), ())),
                                  preferred_element_type=f32)
        ol = (il[:, :, None] == bins8).astype(f32) * wlm.astype(f32)[:, :, None]
        h1 = h1 + jnp.sum(ol.reshape(_CH * 512, 8), axis=0, keepdims=True)
        return h2, h1

    h2, h1 = lax.fori_loop(0, 512 // _CH, chunk,
                           (jnp.zeros((64, 64), f32), jnp.zeros((1, 8), f32)))
    h2 = h2 / jnp.sum(h2)
    h1 = h1 / jnp.sum(h1)

    inv1 = g1_ref[...] * lax.rsqrt(v1_ref[...] + 1e-5)
    sh1 = b1_ref[...] - m1_ref[...] * inv1
    amul = w1t_ref[0:1, :] * inv1
    cadd = jnp.dot(h1, w1t_ref[1:9, :],
                   preferred_element_type=f32) * inv1 + sh1

    def pad2(m, val):
        col = jnp.full((64, 1), val, f32)
        mc = jnp.concatenate([col, m, col], axis=1)
        row = jnp.full((1, 66), val, f32)
        return jnp.concatenate([row, mc, row], axis=0)

    def run3(m, val, op):
        row = jnp.full((1, 66), val, f32)
        mr = jnp.concatenate([row, m, row], axis=0)
        m = op(op(mr[0:66], mr[1:67]), mr[2:68])
        col = jnp.full((66, 1), val, f32)
        mc = jnp.concatenate([col, m, col], axis=1)
        return op(op(mc[:, 0:66], mc[:, 1:67]), mc[:, 2:68])

    mx = run3(pad2(h2, _NEG), _NEG, jnp.maximum)
    mn = run3(pad2(h2, _POS), _POS, jnp.minimum)
    s1 = (lax.broadcasted_iota(jnp.int32, (33, 66), 1)
          == 2 * lax.broadcasted_iota(jnp.int32, (33, 66), 0)).astype(f32)
    s2 = (lax.broadcasted_iota(jnp.int32, (66, 33), 0)
          == 2 * lax.broadcasted_iota(jnp.int32, (66, 33), 1)).astype(f32)
    pmax = jnp.dot(jnp.dot(s1, mx, preferred_element_type=f32), s2,
                   preferred_element_type=f32)
    pmin = jnp.dot(jnp.dot(s1, mn, preferred_element_type=f32), s2,
                   preferred_element_type=f32)

    apos = jnp.maximum(amul, 0.0)
    aneg = jnp.minimum(amul, 0.0)
    astk = jnp.concatenate([apos, aneg], axis=0)
    zs = []
    for j in range(33):
        cols = jnp.concatenate([pmax[:, j:j + 1], pmin[:, j:j + 1]], axis=1)
        zs.append(jnp.dot(cols, astk, preferred_element_type=f32) + cadd)
    zint = jnp.stack(zs, axis=0)

    ii = lax.broadcasted_iota(jnp.int32, (33, 33, 128), 1)
    jj = lax.broadcasted_iota(jnp.int32, (33, 33, 128), 0)
    bflag = (ii == 0) | (ii == 32) | (jj == 0) | (jj == 32)
    sh3 = sh1.reshape(1, 1, 128)
    bterm = jnp.where(bflag, sh3, _NEG)
    z = jnp.maximum(jnp.maximum(zint, bterm), 0.0)

    zb = z.astype(jnp.bfloat16)
    zc0 = jnp.zeros((33, 1, 128), jnp.bfloat16)
    zc = jnp.concatenate([zc0, zb, zc0], axis=1)
    zr0 = jnp.zeros((1, 35, 128), jnp.bfloat16)
    xp = jnp.concatenate([zr0, zc, zr0], axis=0)

    xs = [xp[dj:dj + 33, di:di + 33, :].reshape(33 * 33, 128)
          for di in range(3) for dj in range(3)]
    xcat = jnp.concatenate(xs, axis=1)
    acc = jnp.dot(xcat, w2t_ref[...], preferred_element_type=f32)

    inv2 = g2_ref[...] * lax.rsqrt(v2_ref[...] + 1e-5)
    sh2 = b2_ref[...] - m2_ref[...] * inv2
    y2 = jnp.maximum(acc * inv2 + sh2, 0.0)
    mv = jnp.sum(y2, axis=0, keepdims=True) * (1.0 / 1089.0)
    hv = jnp.dot(mv, wlt_ref[...], preferred_element_type=f32) + bl_ref[...]
    out_ref[...] = hv.reshape(1, 1, 256)


@functools.partial(jax.jit)
def kernel(x, x1, w1, g1, b1, m1, v1, w2, g2, b2, m2, v2, wl, bl):
    B = x1.shape[0]
    w1t = w1.reshape(128, 9).T
    w2t = jnp.transpose(w2, (2, 3, 1, 0)).reshape(9 * 128, 1024)
    w2t = w2t.astype(jnp.bfloat16)
    wlt = wl.T
    row = lambda v: v.reshape(1, -1)

    h = pl.pallas_call(
        _body,
        grid=(B,),
        in_specs=[
            pl.BlockSpec((1, 3, 512, 512), lambda b: (b, 0, 0, 0)),
            pl.BlockSpec((9, 128), lambda b: (0, 0)),
            pl.BlockSpec((1, 128), lambda b: (0, 0)),
            pl.BlockSpec((1, 128), lambda b: (0, 0)),
            pl.BlockSpec((1, 128), lambda b: (0, 0)),
            pl.BlockSpec((1, 128), lambda b: (0, 0)),
            pl.BlockSpec((9 * 128, 1024), lambda b: (0, 0)),
            pl.BlockSpec((1, 1024), lambda b: (0, 0)),
            pl.BlockSpec((1, 1024), lambda b: (0, 0)),
            pl.BlockSpec((1, 1024), lambda b: (0, 0)),
            pl.BlockSpec((1, 1024), lambda b: (0, 0)),
            pl.BlockSpec((1024, 256), lambda b: (0, 0)),
            pl.BlockSpec((1, 256), lambda b: (0, 0)),
        ],
        out_specs=pl.BlockSpec((1, 1, 256), lambda b: (b, 0, 0)),
        out_shape=jax.ShapeDtypeStruct((B, 1, 256), jnp.float32),
        compiler_params=pltpu.CompilerParams(
            dimension_semantics=("parallel",)),
    )(x1, w1t, row(g1), row(b1), row(m1), row(v1),
      w2t, row(g2), row(b2), row(m2), row(v2), wlt, row(bl))
    return jnp.concatenate([x, h.reshape(B, 256)], axis=1)


# speedup vs baseline: 1.0357x; 1.0076x over previous
import functools

import jax
import jax.numpy as jnp
from jax import lax
from jax.experimental import pallas as pl
from jax.experimental.pallas import tpu as pltpu

_CH = 64
_NEG = -1e30
_POS = 1e30


def _body(x1_ref, w1t_ref, g1_ref, b1_ref, m1_ref, v1_ref,
          w2t_ref, g2_ref, b2_ref, m2_ref, v2_ref, wlt_ref, bl_ref,
          out_ref):
    f32 = jnp.float32

    def chunk(i, hall):
        l = x1_ref[0, 0, pl.ds(i * _CH, _CH), :]
        a = x1_ref[0, 1, pl.ds(i * _CH, _CH), :]
        b = x1_ref[0, 2, pl.ds(i * _CH, _CH), :]
        va = (a + 1.0) * 0.5
        vb = (b + 1.0) * 0.5
        vl = (l + 1.0) * 0.5
        wab = ((a != 0) & (b != 0) & (va >= 0) & (va <= 1)
               & (vb >= 0) & (vb <= 1)).astype(jnp.bfloat16)
        wlm = ((l != 0) & (vl >= 0) & (vl <= 1)).astype(jnp.bfloat16)
        ia = jnp.clip(jnp.floor(va * 64.0).astype(jnp.int32), 0, 63)
        ib = jnp.clip(jnp.floor(vb * 64.0).astype(jnp.int32), 0, 63)
        il = jnp.clip(jnp.floor(vl * 8.0).astype(jnp.int32), 0, 7)
        h2, h1 = hall
        bins64 = lax.broadcasted_iota(jnp.int32, (_CH, 512, 64), 2)
        bins8 = lax.broadcasted_iota(jnp.int32, (_CH, 512, 8), 2)
        oa = (ia[:, :, None] == bins64).astype(jnp.bfloat16)
        ob = (ib[:, :, None] == bins64).astype(jnp.bfloat16) * wab[:, :, None]
        oa2 = oa.reshape(_CH * 512, 64)
        ob2 = ob.reshape(_CH * 512, 64)
        h2 = h2 + lax.dot_general(ob2, oa2, (((0,), (0,)), ((), ())),
                                  preferred_element_type=f32)
        ol = (il[:, :, None] == bins8).astype(f32) * wlm.astype(f32)[:, :, None]
        h1 = h1 + jnp.sum(ol.reshape(_CH * 512, 8), axis=0, keepdims=True)
        return h2, h1

    h2, h1 = lax.fori_loop(0, 512 // _CH, chunk,
                           (jnp.zeros((64, 64), f32), jnp.zeros((1, 8), f32)))
    h2 = h2 / jnp.sum(h2)
    h1 = h1 / jnp.sum(h1)

    inv1 = g1_ref[...] * lax.rsqrt(v1_ref[...] + 1e-5)
    sh1 = b1_ref[...] - m1_ref[...] * inv1
    amul = w1t_ref[0:1, :] * inv1
    cadd = jnp.dot(h1, w1t_ref[1:9, :],
                   preferred_element_type=f32) * inv1 + sh1

    def pad2(m, val):
        col = jnp.full((64, 1), val, f32)
        mc = jnp.concatenate([col, m, col], axis=1)
        row = jnp.full((1, 66), val, f32)
        return jnp.concatenate([row, mc, row], axis=0)

    def run3(m, val, op):
        row = jnp.full((1, 66), val, f32)
        mr = jnp.concatenate([row, m, row], axis=0)
        m = op(op(mr[0:66], mr[1:67]), mr[2:68])
        col = jnp.full((66, 1), val, f32)
        mc = jnp.concatenate([col, m, col], axis=1)
        return op(op(mc[:, 0:66], mc[:, 1:67]), mc[:, 2:68])

    mx = run3(pad2(h2, _NEG), _NEG, jnp.maximum)
    mn = run3(pad2(h2, _POS), _POS, jnp.minimum)
    s1 = (lax.broadcasted_iota(jnp.int32, (33, 66), 1)
          == 2 * lax.broadcasted_iota(jnp.int32, (33, 66), 0)).astype(f32)
    s2 = (lax.broadcasted_iota(jnp.int32, (66, 33), 0)
          == 2 * lax.broadcasted_iota(jnp.int32, (66, 33), 1)).astype(f32)
    pmax = jnp.dot(jnp.dot(s1, mx, preferred_element_type=f32), s2,
                   preferred_element_type=f32)
    pmin = jnp.dot(jnp.dot(s1, mn, preferred_element_type=f32), s2,
                   preferred_element_type=f32)

    apos = jnp.maximum(amul, 0.0)
    aneg = jnp.minimum(amul, 0.0)
    astk = jnp.concatenate([apos, aneg], axis=0)
    zs = []
    for j in range(33):
        cols = jnp.concatenate([pmax[:, j:j + 1], pmin[:, j:j + 1]], axis=1)
        zs.append(jnp.dot(cols, astk, preferred_element_type=f32) + cadd)
    zint = jnp.stack(zs, axis=0)

    ii = lax.broadcasted_iota(jnp.int32, (33, 33, 128), 1)
    jj = lax.broadcasted_iota(jnp.int32, (33, 33, 128), 0)
    bflag = (ii == 0) | (ii == 32) | (jj == 0) | (jj == 32)
    sh3 = sh1.reshape(1, 1, 128)
    bterm = jnp.where(bflag, sh3, _NEG)
    z = jnp.maximum(jnp.maximum(zint, bterm), 0.0)

    zb = z.astype(jnp.bfloat16)
    zc0 = jnp.zeros((33, 1, 128), jnp.bfloat16)
    zc = jnp.concatenate([zc0, zb, zc0], axis=1)
    zr0 = jnp.zeros((1, 35, 128), jnp.bfloat16)
    xp = jnp.concatenate([zr0, zc, zr0], axis=0)

    xs = [xp[dj:dj + 33, di:di + 33, :].reshape(33 * 33, 128)
          for di in range(3) for dj in range(3)]
    xcat = jnp.concatenate(xs, axis=1)
    acc = jnp.dot(xcat, w2t_ref[...], preferred_element_type=f32)

    inv2 = g2_ref[...] * lax.rsqrt(v2_ref[...] + 1e-5)
    sh2 = b2_ref[...] - m2_ref[...] * inv2
    y2 = jnp.maximum(acc * inv2 + sh2, 0.0)
    mv = jnp.sum(y2, axis=0, keepdims=True) * (1.0 / 1089.0)
    hv = jnp.dot(mv, wlt_ref[...], preferred_element_type=f32) + bl_ref[...]
    out_ref[...] = hv.reshape(1, 1, 256)


@functools.partial(jax.jit)
def kernel(x, x1, w1, g1, b1, m1, v1, w2, g2, b2, m2, v2, wl, bl):
    B = x1.shape[0]
    w1t = w1.reshape(128, 9).T
    w2t = jnp.transpose(w2, (2, 3, 1, 0)).reshape(9 * 128, 1024)
    w2t = w2t.astype(jnp.bfloat16)
    wlt = wl.T
    row = lambda v: v.reshape(1, -1)

    h = pl.pallas_call(
        _body,
        grid=(B,),
        in_specs=[
            pl.BlockSpec((1, 3, 512, 512), lambda b: (b, 0, 0, 0)),
            pl.BlockSpec((9, 128), lambda b: (0, 0)),
            pl.BlockSpec((1, 128), lambda b: (0, 0)),
            pl.BlockSpec((1, 128), lambda b: (0, 0)),
            pl.BlockSpec((1, 128), lambda b: (0, 0)),
            pl.BlockSpec((1, 128), lambda b: (0, 0)),
            pl.BlockSpec((9 * 128, 1024), lambda b: (0, 0)),
            pl.BlockSpec((1, 1024), lambda b: (0, 0)),
            pl.BlockSpec((1, 1024), lambda b: (0, 0)),
            pl.BlockSpec((1, 1024), lambda b: (0, 0)),
            pl.BlockSpec((1, 1024), lambda b: (0, 0)),
            pl.BlockSpec((1024, 256), lambda b: (0, 0)),
            pl.BlockSpec((1, 256), lambda b: (0, 0)),
        ],
        out_specs=pl.BlockSpec((1, 1, 256), lambda b: (b, 0, 0)),
        out_shape=jax.ShapeDtypeStruct((B, 1, 256), jnp.float32),
        compiler_params=pltpu.CompilerParams(
            dimension_semantics=("parallel",)),
    )(x1, w1t, row(g1), row(b1), row(m1), row(v1),
      w2t, row(g2), row(b2), row(m2), row(v2), wlt, row(bl))
    return jnp.concatenate([x, h.reshape(B, 256)], axis=1)
